# baseline (speedup 1.0000x reference)
import functools

import jax
import jax.numpy as jnp
from jax import lax
from jax.experimental import pallas as pl
from jax.experimental.pallas import tpu as pltpu
from jax.experimental.pallas import tpu_sc as plsc

B, S, H = 32, 8, 1024
V = 100000
ROWS = B * S
NC, NS = 2, 16
NW = NC * NS
RPW = ROWS // NW
WORDS_PER_W = RPW * V
ZN = 16000
NCHUNK = WORDS_PER_W // ZN
MAGIC = 12582912.0

_mesh = plsc.VectorSubcoreMesh(core_axis_name="c", subcore_axis_name="s")


@functools.partial(
    pl.kernel,
    mesh=_mesh,
    out_type=jax.ShapeDtypeStruct((ROWS * V,), jnp.float32),
    scratch_types=[
        pltpu.VMEM((ZN,), jnp.float32),
        pltpu.VMEM((RPW, 128), jnp.float32),
        pltpu.VMEM((16,), jnp.int32),
        pltpu.VMEM((16,), jnp.float32),
        pltpu.SemaphoreType.DMA,
        pltpu.SemaphoreType.DMA,
    ],
)
def _fake_lm_head(hs_hbm, out_hbm, zbuf, hsv, idx_v, val_v, zsem, ssem):
    wid = lax.axis_index("s") * NC + lax.axis_index("c")
    base = wid * RPW

    zero_v = jnp.zeros((16,), jnp.float32)

    def fill(i, c):
        off = i * 128
        for j in range(8):
            zbuf[pl.ds(off + j * 16, 16)] = zero_v
        return c

    lax.fori_loop(0, ZN // 128, fill, 0)

    copies = []
    for k in range(NCHUNK):
        dst = out_hbm.at[pl.ds(base * V + k * ZN, ZN)]
        copies.append(pltpu.async_copy(zbuf, dst, zsem))

    pltpu.sync_copy(hs_hbm.at[pl.ds(base, RPW), pl.ds(0, 128)], hsv)
    lane = lax.iota(jnp.int32, 16)
    acc = jnp.zeros((16,), jnp.int32)
    for r in range(RPW):
        x = hsv[r, pl.ds(0, 16)][0]
        rr = (x + MAGIC) - MAGIC
        rr = jnp.minimum(jnp.maximum(rr, 0.0), float(V - 1))
        tok = rr.astype(jnp.int32)
        flat_r = (base + r) * V + tok
        if r == 0:
            acc = flat_r + acc
        else:
            acc = jnp.where(lane == r, flat_r, acc)
    idx_v[...] = acc
    val_v[...] = jnp.full((16,), 5.0, jnp.float32)

    for c in copies:
        c.wait()

    pltpu.async_copy(val_v, out_hbm.at[idx_v], ssem).wait()


def kernel(hidden_states):
    hs = hidden_states.reshape(ROWS, H)
    out = _fake_lm_head(hs)
    return out.reshape(B, S, V)
